# baseline (speedup 1.0000x reference)
import functools

import jax
import jax.numpy as jnp
from jax import lax
from jax.experimental import pallas as pl
from jax.experimental.pallas import tpu as pltpu
from jax.experimental.pallas import tpu_sc as plsc

N = 10000
E = 320000
D = 128

NC = 2
NS = 16
NW = NC * NS
DH = D // NC
CH = 125
EW = E // NS
KW = EW // CH
EDW = E // NW
RS = N // NS
NB = 4

_mesh = plsc.VectorSubcoreMesh(
    core_axis_name="c", subcore_axis_name="s", num_cores=NC, num_subcores=NS
)


_SC_PARAMS = pltpu.CompilerParams(
    use_tc_tiling_on_sc=False, needs_layout_passes=False
)


@functools.partial(
    pl.kernel,
    out_type=jax.ShapeDtypeStruct((NC, N, DH), jnp.float32),
    mesh=_mesh,
    scratch_types=[
        pltpu.VMEM((KW, CH), jnp.int32),
        pltpu.VMEM((KW, CH), jnp.int32),
    ]
    + [pltpu.VMEM((CH, DH), jnp.float32)] * NB
    + [pltpu.VMEM_SHARED((N, DH), jnp.float32)]
    + [pltpu.SemaphoreType.DMA] * (2 * NB),
    compiler_params=_SC_PARAMS,
)
def _sc_agg(tab_a, tab_b, srcs, dsts, zeros, out_acc, src_v, dst_v, *rest):
    bufs = list(rest[:NB])
    acc_sh = rest[NB]
    gsem = list(rest[NB + 1 : NB + 1 + NB])
    ssem = list(rest[NB + 1 + NB :])
    c = lax.axis_index("c")
    s = lax.axis_index("s")
    r0 = s * RS

    pltpu.sync_copy(zeros.at[pl.ds(r0, RS)], acc_sh.at[pl.ds(r0, RS)])
    pltpu.sync_copy(srcs.at[pl.ds(s * KW, KW)], src_v)
    pltpu.sync_copy(dsts.at[pl.ds(s * KW, KW)], dst_v)
    plsc.subcore_barrier()

    def gather(k, i):
        @pl.when(c == 0)
        def _():
            pltpu.async_copy(tab_a.at[src_v.at[k]], bufs[i], gsem[i])

        @pl.when(c == 1)
        def _():
            pltpu.async_copy(tab_b.at[src_v.at[k]], bufs[i], gsem[i])

    def wait_gather(k, i):
        pltpu.make_async_copy(tab_a.at[src_v.at[k]], bufs[i], gsem[i]).wait()

    def scatter(k, i):
        pltpu.async_copy(bufs[i], acc_sh.at[dst_v.at[k]], ssem[i], add=True)

    def wait_scatter(k, i):
        pltpu.make_async_copy(bufs[i], acc_sh.at[dst_v.at[k]], ssem[i]).wait()

    for i in range(NB):
        gather(i, i)

    def body(g, carry):
        k0 = NB * g
        for i in range(NB):
            wait_gather(k0 + i, i)
            scatter(k0 + i, i)
        for i in range(NB):
            kn = k0 + NB + i

            @pl.when(kn < KW)
            def _():
                wait_scatter(k0 + i, i)
                gather(kn, i)

        return carry

    lax.fori_loop(0, KW // NB, body, 0)
    for i in range(NB):
        wait_scatter(KW - NB + i, i)
    plsc.subcore_barrier()

    pltpu.sync_copy(acc_sh.at[pl.ds(r0, RS)], out_acc.at[c].at[pl.ds(r0, RS)])


@functools.partial(
    pl.kernel,
    out_type=jax.ShapeDtypeStruct((NW, N), jnp.float32),
    mesh=_mesh,
    scratch_types=[
        pltpu.VMEM((EDW,), jnp.int32),
        pltpu.VMEM((N,), jnp.float32),
    ],
    compiler_params=_SC_PARAMS,
)
def _sc_deg(dsts_flat, out_deg, dst_v, deg_v):
    c = lax.axis_index("c")
    s = lax.axis_index("s")
    wid = s * NC + c
    pltpu.sync_copy(dsts_flat.at[wid], dst_v)

    def zero_deg(i, carry):
        deg_v[pl.ds(i * 16, 16)] = jnp.zeros((16,), jnp.float32)
        return carry

    lax.fori_loop(0, N // 16, zero_deg, 0)
    ones16 = jnp.ones((16,), jnp.float32)

    def upd(j, carry):
        plsc.addupdate_scatter(deg_v, [dst_v[pl.ds(j * 16, 16)]], ones16)
        return carry

    lax.fori_loop(0, EDW // 16, upd, 0)
    pltpu.sync_copy(deg_v, out_deg.at[wid])


BN = 1000


def _mean_matmul(h, acc_ref, deg_ref, ws_ref, wn_ref, b_ref, relu):
    agg = jnp.concatenate([acc_ref[0], acc_ref[1]], axis=1)
    inv = 1.0 / jnp.maximum(deg_ref[...], 1.0)
    hn = agg * inv
    o = (
        jnp.dot(h, ws_ref[...], preferred_element_type=jnp.float32)
        + jnp.dot(hn, wn_ref[...], preferred_element_type=jnp.float32)
        + b_ref[...]
    )
    if relu:
        o = jnp.maximum(o, 0.0)
    return o


def _combine1_body(h_ref, acc_ref, deg_ref, ws_ref, wn_ref, b_ref,
                   oa_ref, ob_ref):
    o = _mean_matmul(h_ref[...], acc_ref, deg_ref, ws_ref, wn_ref, b_ref, True)
    oa_ref[...] = o[:, :DH]
    ob_ref[...] = o[:, DH:]


def _combine2_body(ha_ref, hb_ref, acc_ref, deg_ref, ws_ref, wn_ref, b_ref,
                   o_ref):
    h = jnp.concatenate([ha_ref[...], hb_ref[...]], axis=1)
    o_ref[...] = _mean_matmul(h, acc_ref, deg_ref, ws_ref, wn_ref, b_ref, False)


_SPEC_H = pl.BlockSpec((BN, D), lambda i: (i, 0))
_SPEC_HH = pl.BlockSpec((BN, DH), lambda i: (i, 0))
_SPEC_ACC = pl.BlockSpec((2, BN, DH), lambda i: (0, i, 0))
_SPEC_DEG = pl.BlockSpec((BN, 1), lambda i: (i, 0))
_SPEC_W = pl.BlockSpec((D, D), lambda i: (0, 0))
_SPEC_B = pl.BlockSpec((1, D), lambda i: (0, 0))


def _combine1(h, acc, deg, ws, wn, b):
    return pl.pallas_call(
        _combine1_body,
        grid=(N // BN,),
        in_specs=[_SPEC_H, _SPEC_ACC, _SPEC_DEG, _SPEC_W, _SPEC_W, _SPEC_B],
        out_specs=(_SPEC_HH, _SPEC_HH),
        out_shape=(
            jax.ShapeDtypeStruct((N, DH), jnp.float32),
            jax.ShapeDtypeStruct((N, DH), jnp.float32),
        ),
    )(h, acc, deg, ws, wn, b)


def _combine2(ha, hb, acc, deg, ws, wn, b):
    return pl.pallas_call(
        _combine2_body,
        grid=(N // BN,),
        in_specs=[_SPEC_HH, _SPEC_HH, _SPEC_ACC, _SPEC_DEG,
                  _SPEC_W, _SPEC_W, _SPEC_B],
        out_specs=_SPEC_H,
        out_shape=jax.ShapeDtypeStruct((N, D), jnp.float32),
    )(ha, hb, acc, deg, ws, wn, b)


def kernel(x, edge_index, W_self1, W_neigh1, b1, W_self2, W_neigh2, b2):
    src = edge_index[0].reshape(E // CH, CH)
    dst = edge_index[1].reshape(E // CH, CH)
    zeros = jnp.zeros((N, DH), jnp.float32)
    xa = x[:, :DH]
    xb = x[:, DH:]

    deg_parts = _sc_deg(edge_index[1].reshape(NW, EDW))
    acc1 = _sc_agg(xa, xb, src, dst, zeros)
    deg = jnp.sum(deg_parts, axis=0)[:, None]
    ha, hb = _combine1(x, acc1, deg, W_self1, W_neigh1, b1[None, :])
    acc2 = _sc_agg(ha, hb, src, dst, zeros)
    out = _combine2(ha, hb, acc2, deg, W_self2, W_neigh2, b2[None, :])
    return out


# speedup vs baseline: 1.0493x; 1.0493x over previous
import functools

import jax
import jax.numpy as jnp
from jax import lax
from jax.experimental import pallas as pl
from jax.experimental.pallas import tpu as pltpu
from jax.experimental.pallas import tpu_sc as plsc

N = 10000
E = 320000
D = 128

NC = 2
NS = 16
NW = NC * NS
CH = 80
EW = E // NW
KW = EW // CH
EDW = E // NW
RS = N // NS
NB = 3

_mesh = plsc.VectorSubcoreMesh(
    core_axis_name="c", subcore_axis_name="s", num_cores=NC, num_subcores=NS
)


_SC_PARAMS = pltpu.CompilerParams(
    use_tc_tiling_on_sc=False, needs_layout_passes=False
)


@functools.partial(
    pl.kernel,
    out_type=jax.ShapeDtypeStruct((NC, N, D), jnp.float32),
    mesh=_mesh,
    scratch_types=[
        pltpu.VMEM((KW, CH), jnp.int32),
        pltpu.VMEM((KW, CH), jnp.int32),
    ]
    + [pltpu.VMEM((CH, D), jnp.float32)] * NB
    + [pltpu.VMEM_SHARED((N, D), jnp.float32)]
    + [pltpu.SemaphoreType.DMA] * (2 * NB),
    compiler_params=_SC_PARAMS,
)
def _sc_agg(table, srcs, dsts, zeros, out_acc, src_v, dst_v, *rest):
    bufs = list(rest[:NB])
    acc_sh = rest[NB]
    gsem = list(rest[NB + 1 : NB + 1 + NB])
    ssem = list(rest[NB + 1 + NB :])
    c = lax.axis_index("c")
    s = lax.axis_index("s")
    wid = s * NC + c
    r0 = s * RS

    pltpu.sync_copy(zeros.at[pl.ds(r0, RS)], acc_sh.at[pl.ds(r0, RS)])
    pltpu.sync_copy(srcs.at[pl.ds(wid * KW, KW)], src_v)
    pltpu.sync_copy(dsts.at[pl.ds(wid * KW, KW)], dst_v)
    plsc.subcore_barrier()

    def gather(k, i):
        pltpu.async_copy(table.at[src_v.at[k]], bufs[i], gsem[i])

    def wait_gather(k, i):
        pltpu.make_async_copy(table.at[src_v.at[k]], bufs[i], gsem[i]).wait()

    def scatter(k, i):
        pltpu.async_copy(bufs[i], acc_sh.at[dst_v.at[k]], ssem[i], add=True)

    def wait_scatter(k, i):
        pltpu.make_async_copy(bufs[i], acc_sh.at[dst_v.at[k]], ssem[i]).wait()

    for i in range(NB):
        gather(i, i)

    def body(g, carry):
        k0 = NB * g
        for i in range(NB):
            wait_gather(k0 + i, i)
            scatter(k0 + i, i)
        for i in range(NB):
            kn = k0 + NB + i

            @pl.when(kn < KW)
            def _():
                wait_scatter(k0 + i, i)
                gather(kn, i)

        return carry

    nfull = KW // NB
    lax.fori_loop(0, nfull, body, 0)
    for i in range(KW % NB):
        k = nfull * NB + i
        wait_gather(k, i)
        scatter(k, i)
    for i in range(KW % NB, NB):
        wait_scatter((nfull - 1) * NB + i, i)
    for i in range(KW % NB):
        wait_scatter(nfull * NB + i, i)
    plsc.subcore_barrier()

    pltpu.sync_copy(acc_sh.at[pl.ds(r0, RS)], out_acc.at[c].at[pl.ds(r0, RS)])


@functools.partial(
    pl.kernel,
    out_type=jax.ShapeDtypeStruct((NW, N), jnp.float32),
    mesh=_mesh,
    scratch_types=[
        pltpu.VMEM((EDW,), jnp.int32),
        pltpu.VMEM((N,), jnp.float32),
    ],
    compiler_params=_SC_PARAMS,
)
def _sc_deg(dsts_flat, out_deg, dst_v, deg_v):
    c = lax.axis_index("c")
    s = lax.axis_index("s")
    wid = s * NC + c
    pltpu.sync_copy(dsts_flat.at[wid], dst_v)

    def zero_deg(i, carry):
        deg_v[pl.ds(i * 16, 16)] = jnp.zeros((16,), jnp.float32)
        return carry

    lax.fori_loop(0, N // 16, zero_deg, 0)
    ones16 = jnp.ones((16,), jnp.float32)

    def upd(j, carry):
        plsc.addupdate_scatter(deg_v, [dst_v[pl.ds(j * 16, 16)]], ones16)
        return carry

    lax.fori_loop(0, EDW // 16, upd, 0)
    pltpu.sync_copy(deg_v, out_deg.at[wid])


BN = 1000


def _combine_body(relu, h_ref, acc_ref, deg_ref, ws_ref, wn_ref, b_ref, o_ref):
    h = h_ref[...]
    agg = acc_ref[0] + acc_ref[1]
    inv = 1.0 / jnp.maximum(deg_ref[...], 1.0)
    hn = agg * inv
    o = (
        jnp.dot(h, ws_ref[...], preferred_element_type=jnp.float32)
        + jnp.dot(hn, wn_ref[...], preferred_element_type=jnp.float32)
        + b_ref[...]
    )
    if relu:
        o = jnp.maximum(o, 0.0)
    o_ref[...] = o


def _combine(h, acc, deg, ws, wn, b, relu):
    return pl.pallas_call(
        functools.partial(_combine_body, relu),
        grid=(N // BN,),
        in_specs=[
            pl.BlockSpec((BN, D), lambda i: (i, 0)),
            pl.BlockSpec((2, BN, D), lambda i: (0, i, 0)),
            pl.BlockSpec((BN, 1), lambda i: (i, 0)),
            pl.BlockSpec((D, D), lambda i: (0, 0)),
            pl.BlockSpec((D, D), lambda i: (0, 0)),
            pl.BlockSpec((1, D), lambda i: (0, 0)),
        ],
        out_specs=pl.BlockSpec((BN, D), lambda i: (i, 0)),
        out_shape=jax.ShapeDtypeStruct((N, D), jnp.float32),
    )(h, acc, deg, ws, wn, b)


def kernel(x, edge_index, W_self1, W_neigh1, b1, W_self2, W_neigh2, b2):
    src = edge_index[0].reshape(E // CH, CH)
    dst = edge_index[1].reshape(E // CH, CH)
    zeros = jnp.zeros((N, D), jnp.float32)

    deg_parts = _sc_deg(edge_index[1].reshape(NW, EDW))
    acc1 = _sc_agg(x, src, dst, zeros)
    deg = jnp.sum(deg_parts, axis=0)[:, None]
    h1 = _combine(x, acc1, deg, W_self1, W_neigh1, b1[None, :], relu=True)
    acc2 = _sc_agg(h1, src, dst, zeros)
    out = _combine(h1, acc2, deg, W_self2, W_neigh2, b2[None, :], relu=False)
    return out


# speedup vs baseline: 1.0949x; 1.0434x over previous
import functools

import jax
import jax.numpy as jnp
from jax import lax
from jax.experimental import pallas as pl
from jax.experimental.pallas import tpu as pltpu
from jax.experimental.pallas import tpu_sc as plsc

N = 10000
E = 320000
D = 128

NC = 2
NS = 16
NW = NC * NS
CH = 100
EW = E // NW
KW = EW // CH
RS = N // NS

_mesh = plsc.VectorSubcoreMesh(
    core_axis_name="c", subcore_axis_name="s", num_cores=NC, num_subcores=NS
)


_SC_PARAMS = pltpu.CompilerParams(
    use_tc_tiling_on_sc=False, needs_layout_passes=False
)


@functools.partial(
    pl.kernel,
    out_type=jax.ShapeDtypeStruct((NC, N, D), jnp.float32),
    mesh=_mesh,
    scratch_types=[
        pltpu.VMEM((KW, CH), jnp.int32),
        pltpu.VMEM((KW, CH), jnp.int32),
        pltpu.VMEM((CH, D), jnp.float32),
        pltpu.VMEM((CH, D), jnp.float32),
        pltpu.VMEM_SHARED((N, D), jnp.float32),
        pltpu.SemaphoreType.DMA,
        pltpu.SemaphoreType.DMA,
    ],
    compiler_params=_SC_PARAMS,
)
def _sc_agg(table, srcs, dsts, zeros, out_acc,
            src_v, dst_v, buf_a, buf_b, acc_sh, sem_a, sem_b):
    c = lax.axis_index("c")
    s = lax.axis_index("s")
    wid = s * NC + c
    r0 = s * RS

    pltpu.sync_copy(zeros.at[pl.ds(r0, RS)], acc_sh.at[pl.ds(r0, RS)])
    pltpu.sync_copy(srcs.at[pl.ds(wid * KW, KW)], src_v)
    pltpu.sync_copy(dsts.at[pl.ds(wid * KW, KW)], dst_v)
    plsc.subcore_barrier()

    def gather(k, buf, sem):
        pltpu.async_copy(table.at[src_v.at[k]], buf, sem)

    def finish(k, buf, sem):
        pltpu.make_async_copy(table.at[src_v.at[k]], buf, sem).wait()
        pltpu.sync_copy(buf, acc_sh.at[dst_v.at[k]], add=True)

    gather(0, buf_a, sem_a)

    def body(g, carry):
        ka = 2 * g
        gather(ka + 1, buf_b, sem_b)
        finish(ka, buf_a, sem_a)

        @pl.when(ka + 2 < KW)
        def _():
            gather(ka + 2, buf_a, sem_a)

        finish(ka + 1, buf_b, sem_b)
        return carry

    lax.fori_loop(0, KW // 2, body, 0)
    plsc.subcore_barrier()

    pltpu.sync_copy(acc_sh.at[pl.ds(r0, RS)], out_acc.at[c].at[pl.ds(r0, RS)])


@functools.partial(
    pl.kernel,
    out_type=jax.ShapeDtypeStruct((NW, N), jnp.float32),
    mesh=_mesh,
    scratch_types=[
        pltpu.VMEM((EW,), jnp.int32),
        pltpu.VMEM((N,), jnp.float32),
    ],
    compiler_params=_SC_PARAMS,
)
def _sc_deg(edges, out_deg, dst_v, deg_v):
    c = lax.axis_index("c")
    s = lax.axis_index("s")
    wid = s * NC + c
    pltpu.sync_copy(edges.at[1, pl.ds(wid * EW, EW)], dst_v)

    def zero_deg(i, carry):
        deg_v[pl.ds(i * 16, 16)] = jnp.zeros((16,), jnp.float32)
        return carry

    lax.fori_loop(0, N // 16, zero_deg, 0)
    ones16 = jnp.ones((16,), jnp.float32)

    def upd(j, carry):
        plsc.addupdate_scatter(deg_v, [dst_v[pl.ds(j * 16, 16)]], ones16)
        return carry

    lax.fori_loop(0, EW // 16, upd, 0)
    pltpu.sync_copy(deg_v, out_deg.at[wid])


BN = 1000


def _combine_body(relu, h_ref, acc_ref, deg_ref, ws_ref, wn_ref, b_ref, o_ref):
    h = h_ref[...]
    agg = acc_ref[0] + acc_ref[1]
    inv = 1.0 / jnp.maximum(deg_ref[...], 1.0)
    hn = agg * inv
    o = (
        jnp.dot(h, ws_ref[...], preferred_element_type=jnp.float32)
        + jnp.dot(hn, wn_ref[...], preferred_element_type=jnp.float32)
        + b_ref[...]
    )
    if relu:
        o = jnp.maximum(o, 0.0)
    o_ref[...] = o


def _combine(h, acc, deg, ws, wn, b, relu):
    return pl.pallas_call(
        functools.partial(_combine_body, relu),
        grid=(N // BN,),
        in_specs=[
            pl.BlockSpec((BN, D), lambda i: (i, 0)),
            pl.BlockSpec((2, BN, D), lambda i: (0, i, 0)),
            pl.BlockSpec((BN, 1), lambda i: (i, 0)),
            pl.BlockSpec((D, D), lambda i: (0, 0)),
            pl.BlockSpec((D, D), lambda i: (0, 0)),
            pl.BlockSpec((1, D), lambda i: (0, 0)),
        ],
        out_specs=pl.BlockSpec((BN, D), lambda i: (i, 0)),
        out_shape=jax.ShapeDtypeStruct((N, D), jnp.float32),
    )(h, acc, deg, ws, wn, b)


def kernel(x, edge_index, W_self1, W_neigh1, b1, W_self2, W_neigh2, b2):
    src = edge_index[0].reshape(E // CH, CH)
    dst = edge_index[1].reshape(E // CH, CH)
    zeros = jnp.zeros((N, D), jnp.float32)

    deg_parts = _sc_deg(edge_index)
    acc1 = _sc_agg(x, src, dst, zeros)
    deg = jnp.sum(deg_parts, axis=0)[:, None]
    h1 = _combine(x, acc1, deg, W_self1, W_neigh1, b1[None, :], relu=True)
    acc2 = _sc_agg(h1, src, dst, zeros)
    out = _combine(h1, acc2, deg, W_self2, W_neigh2, b2[None, :], relu=False)
    return out


# speedup vs baseline: 1.1101x; 1.0139x over previous
import functools

import jax
import jax.numpy as jnp
from jax import lax
from jax.experimental import pallas as pl
from jax.experimental.pallas import tpu as pltpu
from jax.experimental.pallas import tpu_sc as plsc

N = 10000
E = 320000
D = 128

NC = 2
NS = 16
NW = NC * NS
CH = 100
EW = E // NW
KW = EW // CH
RS = N // NS

_mesh = plsc.VectorSubcoreMesh(
    core_axis_name="c", subcore_axis_name="s", num_cores=NC, num_subcores=NS
)


_SC_PARAMS = pltpu.CompilerParams(
    use_tc_tiling_on_sc=False, needs_layout_passes=False
)


@functools.partial(
    pl.kernel,
    out_type=jax.ShapeDtypeStruct((NC, N, D), jnp.float32),
    mesh=_mesh,
    scratch_types=[
        pltpu.VMEM((KW, CH), jnp.int32),
        pltpu.VMEM((KW, CH), jnp.int32),
        pltpu.VMEM((CH, D), jnp.float32),
        pltpu.VMEM((CH, D), jnp.float32),
        pltpu.VMEM_SHARED((N, D), jnp.float32),
        pltpu.SemaphoreType.DMA,
        pltpu.SemaphoreType.DMA,
    ],
    compiler_params=_SC_PARAMS,
)
def _sc_agg(table, srcs, dsts, zeros, out_acc,
            src_v, dst_v, buf_a, buf_b, acc_sh, sem_a, sem_b):
    c = lax.axis_index("c")
    s = lax.axis_index("s")
    wid = s * NC + c
    r0 = s * RS

    pltpu.sync_copy(zeros.at[pl.ds(r0, RS)], acc_sh.at[pl.ds(r0, RS)])
    pltpu.sync_copy(srcs.at[pl.ds(wid * KW, KW)], src_v)
    pltpu.sync_copy(dsts.at[pl.ds(wid * KW, KW)], dst_v)
    plsc.subcore_barrier()

    def gather(k, buf, sem):
        pltpu.async_copy(table.at[src_v.at[k]], buf, sem)

    def finish(k, buf, sem):
        pltpu.make_async_copy(table.at[src_v.at[k]], buf, sem).wait()
        pltpu.sync_copy(buf, acc_sh.at[dst_v.at[k]], add=True)

    gather(0, buf_a, sem_a)

    def body(g, carry):
        ka = 2 * g
        gather(ka + 1, buf_b, sem_b)
        finish(ka, buf_a, sem_a)

        @pl.when(ka + 2 < KW)
        def _():
            gather(ka + 2, buf_a, sem_a)

        finish(ka + 1, buf_b, sem_b)
        return carry

    lax.fori_loop(0, KW // 2, body, 0)
    plsc.subcore_barrier()

    pltpu.sync_copy(acc_sh.at[pl.ds(r0, RS)], out_acc.at[c].at[pl.ds(r0, RS)])


@functools.partial(
    pl.kernel,
    out_type=jax.ShapeDtypeStruct((N // 1000, NW, 1000), jnp.float32),
    mesh=_mesh,
    scratch_types=[
        pltpu.VMEM((EW,), jnp.int32),
        pltpu.VMEM((N,), jnp.float32),
    ],
    compiler_params=_SC_PARAMS,
)
def _sc_deg(dsts_flat, out_deg, dst_v, deg_v):
    c = lax.axis_index("c")
    s = lax.axis_index("s")
    wid = s * NC + c
    pltpu.sync_copy(dsts_flat.at[wid], dst_v)

    def zero_deg(i, carry):
        deg_v[pl.ds(i * 16, 16)] = jnp.zeros((16,), jnp.float32)
        return carry

    lax.fori_loop(0, N // 16, zero_deg, 0)
    ones16 = jnp.ones((16,), jnp.float32)

    def upd(j, carry):
        plsc.addupdate_scatter(deg_v, [dst_v[pl.ds(j * 16, 16)]], ones16)
        return carry

    lax.fori_loop(0, EW // 16, upd, 0)
    for blk in range(N // 1000):
        pltpu.sync_copy(deg_v.at[pl.ds(blk * 1000, 1000)], out_deg.at[blk, wid])


BN = 1000


def _fused_out(h, hn_scaled, ws_ref, wn_ref, b_ref, relu):
    o = (
        jnp.dot(h, ws_ref[...], preferred_element_type=jnp.float32)
        + jnp.dot(hn_scaled, wn_ref[...], preferred_element_type=jnp.float32)
        + b_ref[...]
    )
    if relu:
        o = jnp.maximum(o, 0.0)
    return o


def _combine1_body(h_ref, acc_ref, dp_ref, ws_ref, wn_ref, b_ref,
                   o_ref, inv_ref):
    deg = jnp.sum(dp_ref[0], axis=0)[:, None]
    inv = 1.0 / jnp.maximum(deg, 1.0)
    inv_ref[...] = inv
    hn = (acc_ref[0] + acc_ref[1]) * inv
    o_ref[...] = _fused_out(h_ref[...], hn, ws_ref, wn_ref, b_ref, True)


def _combine2_body(h_ref, acc_ref, inv_ref, ws_ref, wn_ref, b_ref, o_ref):
    hn = (acc_ref[0] + acc_ref[1]) * inv_ref[...]
    o_ref[...] = _fused_out(h_ref[...], hn, ws_ref, wn_ref, b_ref, False)


_SPEC_H = pl.BlockSpec((BN, D), lambda i: (i, 0))
_SPEC_ACC = pl.BlockSpec((2, BN, D), lambda i: (0, i, 0))
_SPEC_COL = pl.BlockSpec((BN, 1), lambda i: (i, 0))
_SPEC_W = pl.BlockSpec((D, D), lambda i: (0, 0))
_SPEC_B = pl.BlockSpec((1, D), lambda i: (0, 0))


def _combine1(h, acc, deg_parts, ws, wn, b):
    return pl.pallas_call(
        _combine1_body,
        grid=(N // BN,),
        in_specs=[
            _SPEC_H,
            _SPEC_ACC,
            pl.BlockSpec((1, NW, BN), lambda i: (i, 0, 0)),
            _SPEC_W,
            _SPEC_W,
            _SPEC_B,
        ],
        out_specs=(_SPEC_H, _SPEC_COL),
        out_shape=(
            jax.ShapeDtypeStruct((N, D), jnp.float32),
            jax.ShapeDtypeStruct((N, 1), jnp.float32),
        ),
    )(h, acc, deg_parts, ws, wn, b)


def _combine2(h, acc, inv, ws, wn, b):
    return pl.pallas_call(
        _combine2_body,
        grid=(N // BN,),
        in_specs=[_SPEC_H, _SPEC_ACC, _SPEC_COL, _SPEC_W, _SPEC_W, _SPEC_B],
        out_specs=_SPEC_H,
        out_shape=jax.ShapeDtypeStruct((N, D), jnp.float32),
    )(h, acc, inv, ws, wn, b)


def kernel(x, edge_index, W_self1, W_neigh1, b1, W_self2, W_neigh2, b2):
    src = edge_index[0].reshape(E // CH, CH)
    dst = edge_index[1].reshape(E // CH, CH)
    zeros = jnp.zeros((N, D), jnp.float32)

    deg_parts = _sc_deg(edge_index[1].reshape(NW, EW))
    acc1 = _sc_agg(x, src, dst, zeros)
    h1, inv = _combine1(x, acc1, deg_parts, W_self1, W_neigh1, b1[None, :])
    acc2 = _sc_agg(h1, src, dst, zeros)
    out = _combine2(h1, acc2, inv, W_self2, W_neigh2, b2[None, :])
    return out


# speedup vs baseline: 1.1184x; 1.0074x over previous
import functools

import jax
import jax.numpy as jnp
from jax import lax
from jax.experimental import pallas as pl
from jax.experimental.pallas import tpu as pltpu
from jax.experimental.pallas import tpu_sc as plsc

N = 10000
E = 320000
D = 128

NC = 2
NS = 16
NW = NC * NS
CH = 100
EW = E // NW
KW = EW // CH
RS = N // NS

_mesh = plsc.VectorSubcoreMesh(
    core_axis_name="c", subcore_axis_name="s", num_cores=NC, num_subcores=NS
)


_SC_PARAMS = pltpu.CompilerParams(
    use_tc_tiling_on_sc=False, needs_layout_passes=False
)


@functools.partial(
    pl.kernel,
    out_type=jax.ShapeDtypeStruct((NC, N, D), jnp.float32),
    mesh=_mesh,
    scratch_types=[
        pltpu.VMEM((KW, CH), jnp.int32),
        pltpu.VMEM((KW, CH), jnp.int32),
        pltpu.VMEM((CH, D), jnp.float32),
        pltpu.VMEM((CH, D), jnp.float32),
        pltpu.VMEM_SHARED((N, D), jnp.float32),
        pltpu.SemaphoreType.DMA,
        pltpu.SemaphoreType.DMA,
    ],
    compiler_params=_SC_PARAMS,
)
def _sc_agg(table, srcs, dsts, out_acc,
            src_v, dst_v, buf_a, buf_b, acc_sh, sem_a, sem_b):
    c = lax.axis_index("c")
    s = lax.axis_index("s")
    wid = s * NC + c
    r0 = s * RS

    z16 = jnp.zeros((16,), jnp.float32)

    def zrow(i, carry):
        buf_a[i // (D // 16), pl.ds((i % (D // 16)) * 16, 16)] = z16
        return carry

    lax.fori_loop(0, CH * (D // 16), zrow, 0)
    for t in range(RS // CH):
        pltpu.sync_copy(buf_a, acc_sh.at[pl.ds(r0 + t * CH, CH)])
    if RS % CH:
        pltpu.sync_copy(
            buf_a.at[pl.ds(0, RS % CH)],
            acc_sh.at[pl.ds(r0 + (RS // CH) * CH, RS % CH)],
        )
    pltpu.sync_copy(srcs.at[pl.ds(wid * KW, KW)], src_v)
    pltpu.sync_copy(dsts.at[pl.ds(wid * KW, KW)], dst_v)
    plsc.subcore_barrier()

    def gather(k, buf, sem):
        pltpu.async_copy(table.at[src_v.at[k]], buf, sem)

    def finish(k, buf, sem):
        pltpu.make_async_copy(table.at[src_v.at[k]], buf, sem).wait()
        pltpu.sync_copy(buf, acc_sh.at[dst_v.at[k]], add=True)

    gather(0, buf_a, sem_a)

    def body(g, carry):
        ka = 2 * g
        gather(ka + 1, buf_b, sem_b)
        finish(ka, buf_a, sem_a)

        @pl.when(ka + 2 < KW)
        def _():
            gather(ka + 2, buf_a, sem_a)

        finish(ka + 1, buf_b, sem_b)
        return carry

    lax.fori_loop(0, KW // 2, body, 0)
    plsc.subcore_barrier()

    pltpu.sync_copy(acc_sh.at[pl.ds(r0, RS)], out_acc.at[c].at[pl.ds(r0, RS)])


@functools.partial(
    pl.kernel,
    out_type=jax.ShapeDtypeStruct((N // 1000, NW, 1000), jnp.float32),
    mesh=_mesh,
    scratch_types=[
        pltpu.VMEM((EW,), jnp.int32),
        pltpu.VMEM((N,), jnp.float32),
    ],
    compiler_params=_SC_PARAMS,
)
def _sc_deg(dsts_flat, out_deg, dst_v, deg_v):
    c = lax.axis_index("c")
    s = lax.axis_index("s")
    wid = s * NC + c
    pltpu.sync_copy(dsts_flat.at[wid], dst_v)

    def zero_deg(i, carry):
        deg_v[pl.ds(i * 16, 16)] = jnp.zeros((16,), jnp.float32)
        return carry

    lax.fori_loop(0, N // 16, zero_deg, 0)
    ones16 = jnp.ones((16,), jnp.float32)

    def upd(j, carry):
        plsc.addupdate_scatter(deg_v, [dst_v[pl.ds(j * 16, 16)]], ones16)
        return carry

    lax.fori_loop(0, EW // 16, upd, 0)
    for blk in range(N // 1000):
        pltpu.sync_copy(deg_v.at[pl.ds(blk * 1000, 1000)], out_deg.at[blk, wid])


BN = 1000


def _fused_out(h, hn_scaled, ws_ref, wn_ref, b_ref, relu):
    o = (
        jnp.dot(h, ws_ref[...], preferred_element_type=jnp.float32)
        + jnp.dot(hn_scaled, wn_ref[...], preferred_element_type=jnp.float32)
        + b_ref[...]
    )
    if relu:
        o = jnp.maximum(o, 0.0)
    return o


def _combine1_body(h_ref, acc_ref, dp_ref, ws_ref, wn_ref, b_ref,
                   o_ref, inv_ref):
    deg = jnp.sum(dp_ref[0], axis=0)[:, None]
    inv = 1.0 / jnp.maximum(deg, 1.0)
    inv_ref[...] = inv
    hn = (acc_ref[0] + acc_ref[1]) * inv
    o_ref[...] = _fused_out(h_ref[...], hn, ws_ref, wn_ref, b_ref, True)


def _combine2_body(h_ref, acc_ref, inv_ref, ws_ref, wn_ref, b_ref, o_ref):
    hn = (acc_ref[0] + acc_ref[1]) * inv_ref[...]
    o_ref[...] = _fused_out(h_ref[...], hn, ws_ref, wn_ref, b_ref, False)


_SPEC_H = pl.BlockSpec((BN, D), lambda i: (i, 0))
_SPEC_ACC = pl.BlockSpec((2, BN, D), lambda i: (0, i, 0))
_SPEC_COL = pl.BlockSpec((BN, 1), lambda i: (i, 0))
_SPEC_W = pl.BlockSpec((D, D), lambda i: (0, 0))
_SPEC_B = pl.BlockSpec((1, D), lambda i: (0, 0))


def _combine1(h, acc, deg_parts, ws, wn, b):
    return pl.pallas_call(
        _combine1_body,
        grid=(N // BN,),
        in_specs=[
            _SPEC_H,
            _SPEC_ACC,
            pl.BlockSpec((1, NW, BN), lambda i: (i, 0, 0)),
            _SPEC_W,
            _SPEC_W,
            _SPEC_B,
        ],
        out_specs=(_SPEC_H, _SPEC_COL),
        out_shape=(
            jax.ShapeDtypeStruct((N, D), jnp.float32),
            jax.ShapeDtypeStruct((N, 1), jnp.float32),
        ),
    )(h, acc, deg_parts, ws, wn, b)


def _combine2(h, acc, inv, ws, wn, b):
    return pl.pallas_call(
        _combine2_body,
        grid=(N // BN,),
        in_specs=[_SPEC_H, _SPEC_ACC, _SPEC_COL, _SPEC_W, _SPEC_W, _SPEC_B],
        out_specs=_SPEC_H,
        out_shape=jax.ShapeDtypeStruct((N, D), jnp.float32),
    )(h, acc, inv, ws, wn, b)


def kernel(x, edge_index, W_self1, W_neigh1, b1, W_self2, W_neigh2, b2):
    src = edge_index[0].reshape(E // CH, CH)
    dst = edge_index[1].reshape(E // CH, CH)

    deg_parts = _sc_deg(edge_index[1].reshape(NW, EW))
    acc1 = _sc_agg(x, src, dst)
    h1, inv = _combine1(x, acc1, deg_parts, W_self1, W_neigh1, b1[None, :])
    acc2 = _sc_agg(h1, src, dst)
    out = _combine2(h1, acc2, inv, W_self2, W_neigh2, b2[None, :])
    return out


# speedup vs baseline: 1.1383x; 1.0178x over previous
import functools

import jax
import jax.numpy as jnp
from jax import lax
from jax.experimental import pallas as pl
from jax.experimental.pallas import tpu as pltpu
from jax.experimental.pallas import tpu_sc as plsc

N = 10000
E = 320000
D = 128

NC = 2
NS = 16
NW = NC * NS
CH = 100
EW = E // NW
KW = EW // CH
RS = N // NS

_mesh = plsc.VectorSubcoreMesh(
    core_axis_name="c", subcore_axis_name="s", num_cores=NC, num_subcores=NS
)


_SC_PARAMS = pltpu.CompilerParams(
    use_tc_tiling_on_sc=False, needs_layout_passes=False
)


@functools.partial(
    pl.kernel,
    out_type=jax.ShapeDtypeStruct((NC, N, D), jnp.float32),
    mesh=_mesh,
    scratch_types=[
        pltpu.VMEM((KW, CH), jnp.int32),
        pltpu.VMEM((KW, CH), jnp.int32),
        pltpu.VMEM((CH, D), jnp.float32),
        pltpu.VMEM((CH, D), jnp.float32),
        pltpu.VMEM_SHARED((N, D), jnp.float32),
        pltpu.SemaphoreType.DMA,
        pltpu.SemaphoreType.DMA,
        pltpu.SemaphoreType.DMA,
    ],
    compiler_params=_SC_PARAMS,
)
def _sc_agg(table, srcs, dsts, out_acc,
            src_v, dst_v, buf_a, buf_b, acc_sh, sem_a, sem_b, sem_z):
    c = lax.axis_index("c")
    s = lax.axis_index("s")
    wid = s * NC + c
    r0 = s * RS

    idx_a = pltpu.async_copy(srcs.at[pl.ds(wid * KW, KW)], src_v, sem_a)
    idx_b = pltpu.async_copy(dsts.at[pl.ds(wid * KW, KW)], dst_v, sem_b)
    z16 = jnp.zeros((16,), jnp.float32)

    def zrow(i, carry):
        buf_a[i // (D // 16), pl.ds((i % (D // 16)) * 16, 16)] = z16
        return carry

    lax.fori_loop(0, CH * (D // 16), zrow, 0)
    zcopies = [
        pltpu.async_copy(buf_a, acc_sh.at[pl.ds(r0 + t * CH, CH)], sem_z)
        for t in range(RS // CH)
    ]
    if RS % CH:
        zcopies.append(
            pltpu.async_copy(
                buf_a.at[pl.ds(0, RS % CH)],
                acc_sh.at[pl.ds(r0 + (RS // CH) * CH, RS % CH)],
                sem_z,
            )
        )
    for zc in zcopies:
        zc.wait()
    idx_a.wait()
    idx_b.wait()
    plsc.subcore_barrier()

    def gather(k, buf, sem):
        pltpu.async_copy(table.at[src_v.at[k]], buf, sem)

    def finish(k, buf, sem):
        pltpu.make_async_copy(table.at[src_v.at[k]], buf, sem).wait()
        pltpu.sync_copy(buf, acc_sh.at[dst_v.at[k]], add=True)

    gather(0, buf_a, sem_a)

    def body(g, carry):
        ka = 2 * g
        gather(ka + 1, buf_b, sem_b)
        finish(ka, buf_a, sem_a)

        @pl.when(ka + 2 < KW)
        def _():
            gather(ka + 2, buf_a, sem_a)

        finish(ka + 1, buf_b, sem_b)
        return carry

    lax.fori_loop(0, KW // 2, body, 0)
    plsc.subcore_barrier()

    pltpu.sync_copy(acc_sh.at[pl.ds(r0, RS)], out_acc.at[c].at[pl.ds(r0, RS)])


@functools.partial(
    pl.kernel,
    out_type=jax.ShapeDtypeStruct((N // 1000, NW, 1000), jnp.float32),
    mesh=_mesh,
    scratch_types=[
        pltpu.VMEM((EW,), jnp.int32),
        pltpu.VMEM((N,), jnp.float32),
        pltpu.SemaphoreType.DMA,
    ],
    compiler_params=_SC_PARAMS,
)
def _sc_deg(dsts_flat, out_deg, dst_v, deg_v, sem):
    c = lax.axis_index("c")
    s = lax.axis_index("s")
    wid = s * NC + c
    pltpu.sync_copy(dsts_flat.at[wid], dst_v)

    def zero_deg(i, carry):
        deg_v[pl.ds(i * 16, 16)] = jnp.zeros((16,), jnp.float32)
        return carry

    lax.fori_loop(0, N // 16, zero_deg, 0)
    ones16 = jnp.ones((16,), jnp.float32)

    def upd(j, carry):
        plsc.addupdate_scatter(deg_v, [dst_v[pl.ds(j * 16, 16)]], ones16)
        return carry

    lax.fori_loop(0, EW // 16, upd, 0)
    outs = [
        pltpu.async_copy(
            deg_v.at[pl.ds(blk * 1000, 1000)], out_deg.at[blk, wid], sem
        )
        for blk in range(N // 1000)
    ]
    for o in outs:
        o.wait()


BN = 1000


def _fused_out(h, hn_scaled, ws_ref, wn_ref, b_ref, relu):
    o = (
        jnp.dot(h, ws_ref[...], preferred_element_type=jnp.float32)
        + jnp.dot(hn_scaled, wn_ref[...], preferred_element_type=jnp.float32)
        + b_ref[...]
    )
    if relu:
        o = jnp.maximum(o, 0.0)
    return o


def _combine1_body(h_ref, acc_ref, dp_ref, ws_ref, wn_ref, b_ref,
                   o_ref, inv_ref):
    deg = jnp.sum(dp_ref[0], axis=0)[:, None]
    inv = 1.0 / jnp.maximum(deg, 1.0)
    inv_ref[...] = inv
    hn = (acc_ref[0] + acc_ref[1]) * inv
    o_ref[...] = _fused_out(h_ref[...], hn, ws_ref, wn_ref, b_ref, True)


def _combine2_body(h_ref, acc_ref, inv_ref, ws_ref, wn_ref, b_ref, o_ref):
    hn = (acc_ref[0] + acc_ref[1]) * inv_ref[...]
    o_ref[...] = _fused_out(h_ref[...], hn, ws_ref, wn_ref, b_ref, False)


_SPEC_H = pl.BlockSpec((BN, D), lambda i: (i, 0))
_SPEC_ACC = pl.BlockSpec((2, BN, D), lambda i: (0, i, 0))
_SPEC_COL = pl.BlockSpec((BN, 1), lambda i: (i, 0))
_SPEC_W = pl.BlockSpec((D, D), lambda i: (0, 0))
_SPEC_B = pl.BlockSpec((1, D), lambda i: (0, 0))


def _combine1(h, acc, deg_parts, ws, wn, b):
    return pl.pallas_call(
        _combine1_body,
        grid=(N // BN,),
        in_specs=[
            _SPEC_H,
            _SPEC_ACC,
            pl.BlockSpec((1, NW, BN), lambda i: (i, 0, 0)),
            _SPEC_W,
            _SPEC_W,
            _SPEC_B,
        ],
        out_specs=(_SPEC_H, _SPEC_COL),
        out_shape=(
            jax.ShapeDtypeStruct((N, D), jnp.float32),
            jax.ShapeDtypeStruct((N, 1), jnp.float32),
        ),
    )(h, acc, deg_parts, ws, wn, b)


def _combine2(h, acc, inv, ws, wn, b):
    return pl.pallas_call(
        _combine2_body,
        grid=(N // BN,),
        in_specs=[_SPEC_H, _SPEC_ACC, _SPEC_COL, _SPEC_W, _SPEC_W, _SPEC_B],
        out_specs=_SPEC_H,
        out_shape=jax.ShapeDtypeStruct((N, D), jnp.float32),
    )(h, acc, inv, ws, wn, b)


def kernel(x, edge_index, W_self1, W_neigh1, b1, W_self2, W_neigh2, b2):
    src = edge_index[0].reshape(E // CH, CH)
    dst = edge_index[1].reshape(E // CH, CH)

    deg_parts = _sc_deg(edge_index[1].reshape(NW, EW))
    acc1 = _sc_agg(x, src, dst)
    h1, inv = _combine1(x, acc1, deg_parts, W_self1, W_neigh1, b1[None, :])
    acc2 = _sc_agg(h1, src, dst)
    out = _combine2(h1, acc2, inv, W_self2, W_neigh2, b2[None, :])
    return out
